# baseline (speedup 1.0000x reference)
import math

import jax
import jax.numpy as jnp
import numpy as np
from jax import lax
from jax.experimental import pallas as pl
from jax.experimental.pallas import tpu as pltpu
from jax.experimental.pallas import tpu_sc as plsc

VOCAB = 100000
D_MODEL = 768
BATCH = 4
SEQ_LEN = 8192

_NC = 2
_NS = 16
_NW = _NC * _NS
_PPW = SEQ_LEN // _NW
_C = 16
_NJ = _PPW // _C
_LANES = D_MODEL // 16
_SCALE = math.sqrt(float(D_MODEL))


def _build_pe(seq_len, d_model):
    position = np.arange(seq_len, dtype=np.float32)[:, None]
    div_term = np.exp(
        np.arange(0, d_model, 2, dtype=np.float32) * (-np.log(10000.0) / d_model)
    )
    pe = np.zeros((seq_len, d_model), dtype=np.float32)
    pe[:, 0::2] = np.sin(position * div_term)
    pe[:, 1::2] = np.cos(position * div_term)
    f32b = pe.view(np.uint32)
    bits = ((f32b + 0x7FFF + ((f32b >> 16) & 1)) >> 16).astype(np.uint16)
    bits = bits.reshape(seq_len, d_model // 32, 2, 16).astype(np.uint32)
    words = bits[:, :, 0, :] | (bits[:, :, 1, :] << 16)
    words = words.reshape(seq_len, d_model // 2).view(np.int32)
    return jnp.asarray(words)


def _sc_body(ids_hbm, table_hbm, pe_hbm, out_hbm, idx_v,
             r00, r01, r02, r03, r10, r11, r12, r13, p0, p1,
             sg0, sg1, so0, so1, sp0, sp1):
    rbuf = [[r00, r01, r02, r03], [r10, r11, r12, r13]]
    pbuf = [p0, p1]
    sg = [sg0, sg1]
    so = [so0, so1]
    sp = [sp0, sp1]

    wid = lax.axis_index("s") * _NC + lax.axis_index("c")
    pos0 = wid * _PPW

    for b in range(BATCH):
        pltpu.async_copy(
            ids_hbm.at[b, pl.ds(pos0, _PPW)],
            idx_v.at[pl.ds(b * _PPW, _PPW)],
            sp0,
        )
    pltpu.make_async_copy(
        ids_hbm.at[0, pl.ds(0, BATCH * _PPW)], idx_v, sp0
    ).wait()

    def start_chunk(j, ring):
        pltpu.async_copy(
            pe_hbm.at[pl.ds(pos0 + j * _C, _C), :], pbuf[ring], sp[ring]
        )
        for b in range(BATCH):
            pltpu.async_copy(
                table_hbm.at[idx_v.at[pl.ds(b * _PPW + j * _C, _C)]],
                rbuf[ring][b], sg[ring],
            )

    def wait_chunk(ring):
        pltpu.make_async_copy(
            pe_hbm.at[pl.ds(0, _C), :], pbuf[ring], sp[ring]
        ).wait()
        for b in range(BATCH):
            pltpu.make_async_copy(
                table_hbm.at[idx_v.at[pl.ds(0, _C)]], rbuf[ring][b], sg[ring]
            ).wait()

    def wait_outs(ring):
        for b in range(BATCH):
            pltpu.make_async_copy(
                rbuf[ring][b], out_hbm.at[0, pl.ds(0, _C), :], so[ring]
            ).wait()

    start_chunk(0, 0)

    @pl.loop(0, _NJ, step=2)
    def _(jj):
        for ring in range(2):
            j = jj + ring
            nring = (ring + 1) % 2

            @pl.when(j + 1 < _NJ)
            def _():
                @pl.when(j >= 1)
                def _():
                    wait_outs(nring)

                start_chunk(j + 1, nring)

            wait_chunk(ring)

            @pl.loop(0, _C, unroll=2)
            def _(r):
                for k in range(_LANES // 2):
                    w = pbuf[ring][r, pl.ds(k * 16, 16)]
                    pa = lax.bitcast_convert_type(w << 16, jnp.float32)
                    pb = lax.bitcast_convert_type(
                        w & jnp.int32(-65536), jnp.float32
                    )

                    sla = pl.ds(k * 32, 16)
                    slb = pl.ds(k * 32 + 16, 16)
                    for b in range(BATCH):
                        rbuf[ring][b][r, sla] = rbuf[ring][b][r, sla] * _SCALE + pa
                        rbuf[ring][b][r, slb] = rbuf[ring][b][r, slb] * _SCALE + pb

            for b in range(BATCH):
                pltpu.async_copy(
                    rbuf[ring][b],
                    out_hbm.at[b, pl.ds(pos0 + j * _C, _C), :],
                    so[ring],
                )

    wait_outs(0)
    wait_outs(1)


def _combined_embedding(input_ids, token_table, pe):
    mesh = plsc.VectorSubcoreMesh(core_axis_name="c", subcore_axis_name="s")
    fn = pl.kernel(
        _sc_body,
        out_type=jax.ShapeDtypeStruct((BATCH, SEQ_LEN, D_MODEL), jnp.float32),
        mesh=mesh,
        scratch_types=[
            pltpu.VMEM((BATCH * _PPW,), jnp.int32),
        ] + [pltpu.VMEM((_C, D_MODEL), jnp.float32)] * 8
          + [pltpu.VMEM((_C, D_MODEL // 2), jnp.int32)] * 2
          + [pltpu.SemaphoreType.DMA] * 6,
    )
    return fn(input_ids, token_table, pe)


_JIT_DEFAULT = jax.jit(_combined_embedding)

_PE_CACHE = []


def _pe_device():
    if not _PE_CACHE:
        _PE_CACHE.append(_build_pe(SEQ_LEN, D_MODEL))
    return _PE_CACHE[0]


def kernel(input_ids, token_table):
    return _JIT_DEFAULT(input_ids.astype(jnp.int32), token_table, _pe_device())


# speedup vs baseline: 1.0613x; 1.0613x over previous
import math

import jax
import jax.numpy as jnp
import numpy as np
from jax import lax
from jax.experimental import pallas as pl
from jax.experimental.pallas import tpu as pltpu
from jax.experimental.pallas import tpu_sc as plsc

VOCAB = 100000
D_MODEL = 768
BATCH = 4
SEQ_LEN = 8192

_NC = 2
_NS = 16
_NW = _NC * _NS
_PPW = SEQ_LEN // _NW
_C = 16
_NJ = _PPW // _C
_LANES = D_MODEL // 16
_SCALE = math.sqrt(float(D_MODEL))


def _build_pe(seq_len, d_model):
    position = np.arange(seq_len, dtype=np.float32)[:, None]
    div_term = np.exp(
        np.arange(0, d_model, 2, dtype=np.float32) * (-np.log(10000.0) / d_model)
    )
    pe = np.zeros((seq_len, d_model), dtype=np.float32)
    pe[:, 0::2] = np.sin(position * div_term)
    pe[:, 1::2] = np.cos(position * div_term)
    return jnp.asarray(pe)


def _sc_body(ids_hbm, table_hbm, pe_hbm, out_hbm, idx_v,
             r00, r01, r02, r03, r10, r11, r12, r13, p0, p1,
             sg0, sg1, so0, so1, sp0, sp1):
    rbuf = [[r00, r01, r02, r03], [r10, r11, r12, r13]]
    pbuf = [p0, p1]
    sg = [sg0, sg1]
    so = [so0, so1]
    sp = [sp0, sp1]

    wid = lax.axis_index("s") * _NC + lax.axis_index("c")
    pos0 = wid * _PPW

    for b in range(BATCH):
        pltpu.async_copy(
            ids_hbm.at[b, pl.ds(pos0, _PPW)],
            idx_v.at[pl.ds(b * _PPW, _PPW)],
            sp0,
        )
    pltpu.make_async_copy(
        ids_hbm.at[0, pl.ds(0, BATCH * _PPW)], idx_v, sp0
    ).wait()

    def start_chunk(j, ring):
        pltpu.async_copy(
            pe_hbm.at[pl.ds(pos0 + j * _C, _C), :], pbuf[ring], sp[ring]
        )
        for b in range(BATCH):
            pltpu.async_copy(
                table_hbm.at[idx_v.at[pl.ds(b * _PPW + j * _C, _C)]],
                rbuf[ring][b], sg[ring],
            )

    def wait_chunk(ring):
        pltpu.make_async_copy(
            pe_hbm.at[pl.ds(0, _C), :], pbuf[ring], sp[ring]
        ).wait()
        for b in range(BATCH):
            pltpu.make_async_copy(
                table_hbm.at[idx_v.at[pl.ds(0, _C)]], rbuf[ring][b], sg[ring]
            ).wait()

    def wait_outs(ring):
        for b in range(BATCH):
            pltpu.make_async_copy(
                rbuf[ring][b], out_hbm.at[0, pl.ds(0, _C), :], so[ring]
            ).wait()

    start_chunk(0, 0)

    @pl.loop(0, _NJ, step=2)
    def _(jj):
        for ring in range(2):
            j = jj + ring
            nring = (ring + 1) % 2

            @pl.when(j + 1 < _NJ)
            def _():
                @pl.when(j >= 1)
                def _():
                    wait_outs(nring)

                start_chunk(j + 1, nring)

            wait_chunk(ring)

            @pl.loop(0, _C, unroll=2)
            def _(r):
                for g in range(_LANES):
                    sl = pl.ds(g * 16, 16)
                    pv = pbuf[ring][r, sl]
                    for b in range(BATCH):
                        rbuf[ring][b][r, sl] = rbuf[ring][b][r, sl] * _SCALE + pv

            for b in range(BATCH):
                pltpu.async_copy(
                    rbuf[ring][b],
                    out_hbm.at[b, pl.ds(pos0 + j * _C, _C), :],
                    so[ring],
                )

    wait_outs(0)
    wait_outs(1)


def _combined_embedding(input_ids, token_table, pe):
    mesh = plsc.VectorSubcoreMesh(core_axis_name="c", subcore_axis_name="s")
    fn = pl.kernel(
        _sc_body,
        out_type=jax.ShapeDtypeStruct((BATCH, SEQ_LEN, D_MODEL), jnp.float32),
        mesh=mesh,
        scratch_types=[
            pltpu.VMEM((BATCH * _PPW,), jnp.int32),
        ] + [pltpu.VMEM((_C, D_MODEL), jnp.float32)] * 8
          + [pltpu.VMEM((_C, D_MODEL), jnp.float32)] * 2
          + [pltpu.SemaphoreType.DMA] * 6,
    )
    return fn(input_ids, token_table, pe)


_JIT_DEFAULT = jax.jit(_combined_embedding)

_PE_CACHE = []


def _pe_device():
    if not _PE_CACHE:
        _PE_CACHE.append(_build_pe(SEQ_LEN, D_MODEL))
    return _PE_CACHE[0]


def kernel(input_ids, token_table):
    return _JIT_DEFAULT(input_ids.astype(jnp.int32), token_table, _pe_device())
